# baseline (speedup 1.0000x reference)
import functools
import math

import jax
import jax.numpy as jnp
from jax import lax
from jax.experimental import pallas as pl
from jax.experimental.pallas import tpu as pltpu
from jax.experimental.pallas import tpu_sc as plsc

N = 10000
E = 320000
D = 128
HD = 64
H = 64
D_OUT = 7

NC = 2
NS = 16
NW = NC * NS
PE = E // NW
PE2 = E // NS
KC = 80
NCHUNK = PE2 // KC
NPAD = 10240
ROWS_PER_TILE = NPAD // NS
NBUF = 5
NBUFC = 10
CROW = 2 + HD
GRP = 20

_mesh = plsc.VectorSubcoreMesh(core_axis_name="c", subcore_axis_name="s")


@functools.partial(
    pl.kernel,
    out_type=jax.ShapeDtypeStruct((E,), jnp.float32),
    mesh=_mesh,
    scratch_types=[
        pltpu.VMEM((N,), jnp.float32),
        pltpu.VMEM((N,), jnp.float32),
        pltpu.VMEM((N,), jnp.float32),
        pltpu.VMEM((2000,), jnp.int32),
        pltpu.VMEM((2000,), jnp.int32),
        pltpu.VMEM((2000,), jnp.float32),
    ],
    compiler_params=pltpu.CompilerParams(needs_layout_passes=False),
)
def _dist2_sc(px, py, pz, src, dst, d2_out, x_v, y_v, z_v, s_v, d_v, o_v):
    c = lax.axis_index("c")
    s = lax.axis_index("s")
    base = (s * NC + c) * PE
    pltpu.sync_copy(px, x_v)
    pltpu.sync_copy(py, y_v)
    pltpu.sync_copy(pz, z_v)

    def chunk(i, carry):
        off = base + i * 2000
        pltpu.sync_copy(src.at[pl.ds(off, 2000)], s_v)
        pltpu.sync_copy(dst.at[pl.ds(off, 2000)], d_v)

        def grp(j, carry2):
            sl = pl.ds(j * 16, 16)
            si = s_v[sl]
            di = d_v[sl]
            dx = plsc.load_gather(x_v, [si]) - plsc.load_gather(x_v, [di])
            dy = plsc.load_gather(y_v, [si]) - plsc.load_gather(y_v, [di])
            dz = plsc.load_gather(z_v, [si]) - plsc.load_gather(z_v, [di])
            o_v[sl] = dx * dx + dy * dy + dz * dz + 1e-8
            return carry2

        lax.fori_loop(0, 2000 // 16, grp, 0)
        pltpu.sync_copy(o_v, d2_out.at[pl.ds(off, 2000)])
        return carry

    lax.fori_loop(0, PE // 2000, chunk, 0)


@functools.partial(
    pl.kernel,
    out_type=jax.ShapeDtypeStruct((NC, NPAD, HD), jnp.float32),
    mesh=_mesh,
    scratch_types=[pltpu.VMEM((CROW, KC), jnp.int32)] * NBUFC
    + [pltpu.VMEM((KC, HD), jnp.float32)] * NBUF
    + [pltpu.VMEM((KC,), jnp.int32)] * NBUF
    + [pltpu.VMEM_SHARED((NPAD, HD), jnp.float32)]
    + [pltpu.SemaphoreType.DMA] * (NBUFC + 2 * NBUF),
    compiler_params=pltpu.CompilerParams(use_tc_tiling_on_sc=False,
                                         needs_layout_passes=False),
)
def _conv_sc(featH, comb, agg_out, *scr):
    buf = scr[0:NBUFC]
    rows = scr[NBUFC:NBUFC + NBUF]
    dring = scr[NBUFC + NBUF:NBUFC + 2 * NBUF]
    acc_sh = scr[NBUFC + 2 * NBUF]
    sems = scr[NBUFC + 2 * NBUF + 1:]
    sem_c = sems[0:NBUFC]
    sem_g = sems[NBUFC:NBUFC + NBUF]
    sem_s = sems[NBUFC + NBUF:NBUFC + 2 * NBUF]

    c = lax.axis_index("c")
    s = lax.axis_index("s")
    row0 = s * ROWS_PER_TILE
    ft = featH.at[c]
    cbs = comb.at[c].at[s]

    for t in range(NBUFC):
        pltpu.async_copy(cbs.at[t], buf[t], sem_c[t])

    def zrow(j, carry):
        for v in range(HD // 16):
            rows[NBUF - 1][j, pl.ds(v * 16, 16)] = jnp.zeros((16,),
                                                             jnp.float32)
        return carry

    lax.fori_loop(0, KC, zrow, 0)
    for z in range(ROWS_PER_TILE // KC):
        pltpu.sync_copy(rows[NBUF - 1], acc_sh.at[pl.ds(row0 + z * KC, KC)])
    plsc.subcore_barrier()

    for t in range(3):
        pltpu.make_async_copy(cbs.at[t], buf[t], sem_c[t]).wait()
        pltpu.async_copy(ft.at[buf[t].at[0]], rows[t], sem_g[t])

    def outer(g, carry):
        for k in range(NBUFC):
            b = g * NBUFC + k
            i = k % NBUF
            pltpu.make_async_copy(ft.at[buf[k].at[0]], rows[i],
                                  sem_g[i]).wait()

            def mul(gq, carry2):
                for ql in range(GRP):
                    rv = plsc.bitcast(
                        buf[k][2 + 4 * gq + ql // 5,
                               pl.ds(16 * (ql % 5), 16)], jnp.float32)
                    e = 5 * gq + ql // 4
                    sl = pl.ds(16 * (ql % 4), 16)
                    rows[i][e, sl] = rows[i][e, sl] * rv
                return carry2

            lax.fori_loop(0, 4 * KC // GRP, mul, 0)
            for t in range(KC // 16):
                sl = pl.ds(16 * t, 16)
                dring[i][sl] = buf[k][1, sl]
            pltpu.async_copy(rows[i], acc_sh.at[dring[i]], sem_s[i],
                             add=True)

            @pl.when(b + NBUFC < NCHUNK)
            def _issue_comb():
                pltpu.async_copy(cbs.at[b + NBUFC], buf[k], sem_c[k])

            i3 = (i + 3) % NBUF
            k3 = (k + 3) % NBUFC

            @pl.when((b >= 2) & (b + 3 < NCHUNK))
            def _wait_prev_scatter():
                pltpu.make_async_copy(rows[i3], acc_sh.at[dring[i3]],
                                      sem_s[i3]).wait()

            @pl.when(b + 3 < NCHUNK)
            def _issue_gather():
                pltpu.make_async_copy(cbs.at[b + 3], buf[k3],
                                      sem_c[k3]).wait()
                pltpu.async_copy(ft.at[buf[k3].at[0]], rows[i3], sem_g[i3])
        return carry

    lax.fori_loop(0, NCHUNK // NBUFC, outer, 0)
    for i in range(NBUF):
        pltpu.make_async_copy(rows[i], acc_sh.at[dring[i]], sem_s[i]).wait()
    plsc.subcore_barrier()
    pltpu.sync_copy(acc_sh.at[pl.ds(row0, ROWS_PER_TILE)],
                    agg_out.at[c, pl.ds(row0, ROWS_PER_TILE)])


BE = 2000


def _radial_body(d2_ref,
                 a_r1, a_rb, a_r2, b_r1, b_rb, b_r2, c_r1, c_rb, c_r2,
                 oa, ob, oc):
    dist = jnp.sqrt(d2_ref[...])
    for r1, rb, r2, o in ((a_r1, a_rb, a_r2, oa),
                          (b_r1, b_rb, b_r2, ob),
                          (c_r1, c_rb, c_r2, oc)):
        u = dist * r1[...] + rb[...]
        u = u * jax.nn.sigmoid(u)
        r = jnp.dot(u, r2[...], preferred_element_type=jnp.float32)
        o[...] = jnp.stack([r[:, :HD], r[:, HD:]])


_w_spec = pl.BlockSpec((1, H), lambda i: (0, 0))
_w2_spec = pl.BlockSpec((H, D), lambda i: (0, 0))
_radial_tc = pl.pallas_call(
    _radial_body,
    grid=(E // BE,),
    in_specs=[pl.BlockSpec((BE, 1), lambda i: (i, 0))]
    + [_w_spec, _w_spec, _w2_spec] * 3,
    out_specs=[pl.BlockSpec((NC, BE, HD), lambda i: (0, i, 0))] * 3,
    out_shape=[jax.ShapeDtypeStruct((NC, E, HD), jnp.float32)] * 3,
)

BN = 2000


def _make_dense(norm, act, skip, final):
    def body(*refs):
        agg_h, f_h, wm, ws, o = refs
        agg = jnp.concatenate([agg_h[0], agg_h[1]], axis=-1)
        f = jnp.concatenate([f_h[0], f_h[1]], axis=-1)
        out = (jnp.dot(agg, wm[...], preferred_element_type=jnp.float32)
               + jnp.dot(f, ws[...], preferred_element_type=jnp.float32))
        if norm:
            scale = (jnp.sqrt(jnp.sum(out * out, axis=1, keepdims=True))
                     / math.sqrt(D) + 1e-6)
            out = out / scale
        if act:
            out = jnp.maximum(out, 0.0)
        if skip:
            out = out + f
        if final:
            o[...] = out
        else:
            o[...] = jnp.stack([out[:, :HD], out[:, HD:]])

    agg_spec = pl.BlockSpec((NC, BN, HD), lambda i: (0, i, 0))
    fh_spec = pl.BlockSpec((NC, BN, HD), lambda i: (0, i, 0))
    w_spec = pl.BlockSpec((D, D), lambda i: (0, 0))
    if final:
        out_spec = pl.BlockSpec((BN, D), lambda i: (i, 0))
        out_shape = jax.ShapeDtypeStruct((N, D), jnp.float32)
    else:
        out_spec = pl.BlockSpec((NC, BN, HD), lambda i: (0, i, 0))
        out_shape = jax.ShapeDtypeStruct((NC, N, HD), jnp.float32)
    return pl.pallas_call(
        body,
        grid=(N // BN,),
        in_specs=[agg_spec, fh_spec, w_spec, w_spec],
        out_specs=out_spec,
        out_shape=out_shape,
    )


_dense_first = _make_dense(norm=True, act=True, skip=False, final=False)
_dense_mid = _make_dense(norm=True, act=True, skip=True, final=False)
_dense_last = _make_dense(norm=False, act=False, skip=False, final=True)

NUM_LAYERS_S = 8


def kernel(feat, pos, edge_index, W0_r1, W0_rb, W0_r2, W0_msg, W0_self,
           Ws_r1, Ws_rb, Ws_r2, Ws_msg, Ws_self,
           W1_r1, W1_rb, W1_r2, W1_msg, W1_self):
    src = edge_index[0]
    dst = edge_index[1]
    src3 = src.reshape(NS, NCHUNK, KC)
    dst3 = dst.reshape(NS, NCHUNK, KC)
    idxp = jnp.broadcast_to(jnp.stack([src3, dst3], axis=2)[None],
                            (NC, NS, NCHUNK, 2, KC))

    def _pack_comb(rad_h):
        radp = jax.lax.bitcast_convert_type(rad_h, jnp.int32)
        radp = radp.reshape(NC, NS, NCHUNK, HD, KC)
        return jnp.concatenate([idxp, radp], axis=3)

    px = jnp.asarray(pos[:, 0])
    py = jnp.asarray(pos[:, 1])
    pz = jnp.asarray(pos[:, 2])

    d2 = _dist2_sc(px, py, pz, src, dst)
    r0, rs, r1 = _radial_tc(
        d2.reshape(E, 1),
        W0_r1, W0_rb.reshape(1, H), W0_r2,
        Ws_r1, Ws_rb.reshape(1, H), Ws_r2,
        W1_r1, W1_rb.reshape(1, H), W1_r2)

    c0 = _pack_comb(r0)
    cs = _pack_comb(rs)
    c1 = _pack_comb(r1)

    fh = jnp.stack([feat[:, :HD], feat[:, HD:]])
    agg = _conv_sc(fh, c0)
    fh = _dense_first(agg, fh, W0_msg, W0_self)
    for _ in range(NUM_LAYERS_S):
        agg = _conv_sc(fh, cs)
        fh = _dense_mid(agg, fh, Ws_msg, Ws_self)
    agg = _conv_sc(fh, c1)
    w1m = jnp.pad(W1_msg, ((0, 0), (0, D - D_OUT)))
    w1s = jnp.pad(W1_self, ((0, 0), (0, D - D_OUT)))
    out = _dense_last(agg, fh, w1m, w1s)
    return out[:, :D_OUT]


# speedup vs baseline: 1.5896x; 1.5896x over previous
import functools
import math

import jax
import jax.numpy as jnp
from jax import lax
from jax.experimental import pallas as pl
from jax.experimental.pallas import tpu as pltpu
from jax.experimental.pallas import tpu_sc as plsc

N = 10000
E = 320000
D = 128
HD = 64
H = 64
D_OUT = 7

NC = 2
NS = 16
NW = NC * NS
PE = E // NW
PE2 = E // NS
KC = 40
NCHUNK = PE2 // KC
NPAD = 10240
ROWS_PER_TILE = NPAD // NS
NBUF = 10
NRAD = 5
AHEAD = 3
U = 4

_mesh = plsc.VectorSubcoreMesh(core_axis_name="c", subcore_axis_name="s")


@functools.partial(
    pl.kernel,
    out_type=jax.ShapeDtypeStruct((E,), jnp.float32),
    mesh=_mesh,
    scratch_types=[
        pltpu.VMEM((N,), jnp.float32),
        pltpu.VMEM((N,), jnp.float32),
        pltpu.VMEM((N,), jnp.float32),
        pltpu.VMEM((2000,), jnp.int32),
        pltpu.VMEM((2000,), jnp.int32),
        pltpu.VMEM((2000,), jnp.float32),
    ],
    compiler_params=pltpu.CompilerParams(needs_layout_passes=False),
)
def _dist2_sc(px, py, pz, src, dst, d2_out, x_v, y_v, z_v, s_v, d_v, o_v):
    c = lax.axis_index("c")
    s = lax.axis_index("s")
    base = (s * NC + c) * PE
    pltpu.sync_copy(px, x_v)
    pltpu.sync_copy(py, y_v)
    pltpu.sync_copy(pz, z_v)

    def chunk(i, carry):
        off = base + i * 2000
        pltpu.sync_copy(src.at[pl.ds(off, 2000)], s_v)
        pltpu.sync_copy(dst.at[pl.ds(off, 2000)], d_v)

        def grp(j, carry2):
            sl = pl.ds(j * 16, 16)
            si = s_v[sl]
            di = d_v[sl]
            dx = plsc.load_gather(x_v, [si]) - plsc.load_gather(x_v, [di])
            dy = plsc.load_gather(y_v, [si]) - plsc.load_gather(y_v, [di])
            dz = plsc.load_gather(z_v, [si]) - plsc.load_gather(z_v, [di])
            o_v[sl] = dx * dx + dy * dy + dz * dz + 1e-8
            return carry2

        lax.fori_loop(0, 2000 // 16, grp, 0)
        pltpu.sync_copy(o_v, d2_out.at[pl.ds(off, 2000)])
        return carry

    lax.fori_loop(0, PE // 2000, chunk, 0)


@functools.partial(
    pl.kernel,
    out_type=jax.ShapeDtypeStruct((NC, NPAD, HD), jnp.float32),
    mesh=_mesh,
    scratch_types=[
        pltpu.VMEM((NCHUNK, KC), jnp.int32),
        pltpu.VMEM((NCHUNK, KC), jnp.int32),
    ]
    + [pltpu.VMEM((KC, HD), jnp.float32)] * (NBUF + NRAD)
    + [pltpu.VMEM_SHARED((NPAD, HD), jnp.float32)]
    + [pltpu.SemaphoreType.DMA] * (2 * NBUF + NRAD),
    compiler_params=pltpu.CompilerParams(use_tc_tiling_on_sc=False),
)
def _conv_sc(featH, radialH, src3, dst3, agg_out, *scr):
    src_v, dst_v = scr[0], scr[1]
    rows = scr[2:2 + NBUF]
    rad = scr[2 + NBUF:2 + NBUF + NRAD]
    acc_sh = scr[2 + NBUF + NRAD]
    sems = scr[3 + NBUF + NRAD:]
    sem_g = sems[0:NBUF]
    sem_r = sems[NBUF:NBUF + NRAD]
    sem_s = sems[NBUF + NRAD:2 * NBUF + NRAD]

    c = lax.axis_index("c")
    s = lax.axis_index("s")
    base = s * PE2
    row0 = s * ROWS_PER_TILE
    ft = featH.at[c]
    rd = radialH.at[c]

    pltpu.sync_copy(src3.at[s], src_v)
    pltpu.sync_copy(dst3.at[s], dst_v)

    def zrow(j, carry):
        for v in range(HD // 16):
            rows[NBUF - 1][j, pl.ds(v * 16, 16)] = jnp.zeros((16,),
                                                             jnp.float32)
        return carry

    lax.fori_loop(0, KC, zrow, 0)
    for z in range(ROWS_PER_TILE // KC):
        pltpu.sync_copy(rows[NBUF - 1], acc_sh.at[pl.ds(row0 + z * KC, KC)])
    plsc.subcore_barrier()

    def gather_in(b, i, ir_):
        pltpu.async_copy(ft.at[src_v.at[b]], rows[i], sem_g[i])
        pltpu.async_copy(rd.at[pl.ds(base + b * KC, KC)], rad[ir_],
                         sem_r[ir_])

    for i in range(AHEAD):
        gather_in(i, i, i % NRAD)

    LAG = NBUF - AHEAD

    def outer(g, carry):
        for i in range(NBUF):
            b = g * NBUF + i
            ir_ = i % NRAD
            pltpu.make_async_copy(ft.at[src_v.at[b]], rows[i],
                                  sem_g[i]).wait()
            pltpu.make_async_copy(rd.at[pl.ds(base + b * KC, KC)], rad[ir_],
                                  sem_r[ir_]).wait()

            def mul(j, carry2):
                for jj in range(U):
                    for v in range(HD // 16):
                        sl = pl.ds(v * 16, 16)
                        r = U * j + jj
                        rows[i][r, sl] = rows[i][r, sl] * rad[ir_][r, sl]
                return carry2

            lax.fori_loop(0, KC // U, mul, 0)
            pltpu.async_copy(rows[i], acc_sh.at[dst_v.at[b]], sem_s[i],
                             add=True)

            i2 = (i + AHEAD) % NBUF

            @pl.when((b >= LAG) & (b + AHEAD < NCHUNK))
            def _wait_prev():
                pltpu.make_async_copy(rows[i2],
                                      acc_sh.at[dst_v.at[b - LAG]],
                                      sem_s[i2]).wait()

            @pl.when(b + AHEAD < NCHUNK)
            def _issue_next():
                gather_in(b + AHEAD, i2, (i + AHEAD) % NRAD)
        return carry

    lax.fori_loop(0, NCHUNK // NBUF, outer, 0)
    for i in range(NBUF):
        b = NCHUNK - NBUF + i
        pltpu.make_async_copy(rows[i], acc_sh.at[dst_v.at[b]],
                              sem_s[i]).wait()
    plsc.subcore_barrier()
    pltpu.sync_copy(acc_sh.at[pl.ds(row0, ROWS_PER_TILE)],
                    agg_out.at[c, pl.ds(row0, ROWS_PER_TILE)])


BE = 2000


def _radial_body(d2_ref,
                 a_r1, a_rb, a_r2, b_r1, b_rb, b_r2, c_r1, c_rb, c_r2,
                 oa, ob, oc):
    dist = jnp.sqrt(d2_ref[...])
    for r1, rb, r2, o in ((a_r1, a_rb, a_r2, oa),
                          (b_r1, b_rb, b_r2, ob),
                          (c_r1, c_rb, c_r2, oc)):
        u = dist * r1[...] + rb[...]
        u = u * jax.nn.sigmoid(u)
        r = jnp.dot(u, r2[...], preferred_element_type=jnp.float32)
        o[...] = jnp.stack([r[:, :HD], r[:, HD:]])


_w_spec = pl.BlockSpec((1, H), lambda i: (0, 0))
_w2_spec = pl.BlockSpec((H, D), lambda i: (0, 0))
_radial_tc = pl.pallas_call(
    _radial_body,
    grid=(E // BE,),
    in_specs=[pl.BlockSpec((BE, 1), lambda i: (i, 0))]
    + [_w_spec, _w_spec, _w2_spec] * 3,
    out_specs=[pl.BlockSpec((NC, BE, HD), lambda i: (0, i, 0))] * 3,
    out_shape=[jax.ShapeDtypeStruct((NC, E, HD), jnp.float32)] * 3,
)

BN = 2000


def _make_dense(norm, act, skip, final):
    def body(*refs):
        agg_h, f_h, wm, ws, o = refs
        agg = jnp.concatenate([agg_h[0], agg_h[1]], axis=-1)
        f = jnp.concatenate([f_h[0], f_h[1]], axis=-1)
        out = (jnp.dot(agg, wm[...], preferred_element_type=jnp.float32)
               + jnp.dot(f, ws[...], preferred_element_type=jnp.float32))
        if norm:
            scale = (jnp.sqrt(jnp.sum(out * out, axis=1, keepdims=True))
                     / math.sqrt(D) + 1e-6)
            out = out / scale
        if act:
            out = jnp.maximum(out, 0.0)
        if skip:
            out = out + f
        if final:
            o[...] = out
        else:
            o[...] = jnp.stack([out[:, :HD], out[:, HD:]])

    agg_spec = pl.BlockSpec((NC, BN, HD), lambda i: (0, i, 0))
    fh_spec = pl.BlockSpec((NC, BN, HD), lambda i: (0, i, 0))
    w_spec = pl.BlockSpec((D, D), lambda i: (0, 0))
    if final:
        out_spec = pl.BlockSpec((BN, D), lambda i: (i, 0))
        out_shape = jax.ShapeDtypeStruct((N, D), jnp.float32)
    else:
        out_spec = pl.BlockSpec((NC, BN, HD), lambda i: (0, i, 0))
        out_shape = jax.ShapeDtypeStruct((NC, N, HD), jnp.float32)
    return pl.pallas_call(
        body,
        grid=(N // BN,),
        in_specs=[agg_spec, fh_spec, w_spec, w_spec],
        out_specs=out_spec,
        out_shape=out_shape,
    )


_dense_first = _make_dense(norm=True, act=True, skip=False, final=False)
_dense_mid = _make_dense(norm=True, act=True, skip=True, final=False)
_dense_last = _make_dense(norm=False, act=False, skip=False, final=True)

NUM_LAYERS_S = 8


def kernel(feat, pos, edge_index, W0_r1, W0_rb, W0_r2, W0_msg, W0_self,
           Ws_r1, Ws_rb, Ws_r2, Ws_msg, Ws_self,
           W1_r1, W1_rb, W1_r2, W1_msg, W1_self):
    src = edge_index[0]
    dst = edge_index[1]
    src3 = src.reshape(NS, NCHUNK, KC)
    dst3 = dst.reshape(NS, NCHUNK, KC)
    px = jnp.asarray(pos[:, 0])
    py = jnp.asarray(pos[:, 1])
    pz = jnp.asarray(pos[:, 2])

    d2 = _dist2_sc(px, py, pz, src, dst)
    r0, rs, r1 = _radial_tc(
        d2.reshape(E, 1),
        W0_r1, W0_rb.reshape(1, H), W0_r2,
        Ws_r1, Ws_rb.reshape(1, H), Ws_r2,
        W1_r1, W1_rb.reshape(1, H), W1_r2)

    fh = jnp.stack([feat[:, :HD], feat[:, HD:]])
    agg = _conv_sc(fh, r0, src3, dst3)
    fh = _dense_first(agg, fh, W0_msg, W0_self)
    for _ in range(NUM_LAYERS_S):
        agg = _conv_sc(fh, rs, src3, dst3)
        fh = _dense_mid(agg, fh, Ws_msg, Ws_self)
    agg = _conv_sc(fh, r1, src3, dst3)
    w1m = jnp.pad(W1_msg, ((0, 0), (0, D - D_OUT)))
    w1s = jnp.pad(W1_self, ((0, 0), (0, D - D_OUT)))
    out = _dense_last(agg, fh, w1m, w1s)
    return out[:, :D_OUT]
